# baseline (speedup 1.0000x reference)
import functools

import jax
import jax.numpy as jnp
from jax import lax
from jax.experimental import pallas as pl
from jax.experimental.pallas import tpu as pltpu
from jax.experimental.pallas import tpu_sc as plsc

_B, _L = 4096, 200
_N = _B * _L
_D = 32
_V = 1000000
_NC, _NS = 2, 16
_NW = _NC * _NS
_LANES = 16

_mesh = plsc.VectorSubcoreMesh(core_axis_name="c", subcore_axis_name="s")


_TV_PER_W = _V // _NW
_TC_CHUNK = 625
_TC_NCHUNK = _TV_PER_W // _TC_CHUNK
_TC_TOT = _V // _TC_CHUNK


@functools.partial(
    pl.kernel,
    mesh=_mesh,
    compiler_params=pltpu.CompilerParams(use_tc_tiling_on_sc=False, needs_layout_passes=False),
    out_type=jax.ShapeDtypeStruct((_V, _D), jnp.float32),
    scratch_types=[
        pltpu.VMEM((2, _D, _TC_CHUNK), jnp.float32),
        pltpu.VMEM((2, _TC_CHUNK, _D), jnp.float32),
        pltpu.SemaphoreType.DMA((2,)),
        pltpu.SemaphoreType.DMA((2,)),
    ],
)
def _transpose_table(tt_hbm, rm_hbm, src_v, dst_v, isem, osem):
    wid = lax.axis_index("s") * _NC + lax.axis_index("c")
    base = wid * _TV_PER_W
    iota = lax.iota(jnp.int32, _LANES)

    def in_copy(g, slot):
        return pltpu.make_async_copy(
            tt_hbm.at[:, wid * _TC_NCHUNK + g, :],
            src_v.at[slot], isem.at[slot])

    def out_copy(g, slot):
        return pltpu.make_async_copy(
            dst_v.at[slot], rm_hbm.at[pl.ds(base + g * _TC_CHUNK, _TC_CHUNK), :],
            osem.at[slot])

    def transpose_chunk(slot):
        def row_body(i, _):
            v0 = plsc.load_gather(src_v.at[slot], [iota, jnp.full((_LANES,), i, jnp.int32)])
            v1 = plsc.load_gather(src_v.at[slot], [iota + _LANES, jnp.full((_LANES,), i, jnp.int32)])
            dst_v[slot, i, pl.ds(0, _LANES)] = v0
            dst_v[slot, i, pl.ds(_LANES, _LANES)] = v1
            return 0
        lax.fori_loop(0, _TC_CHUNK, row_body, 0, unroll=4)

    in_copy(0, 0).start()
    in_copy(1, 1).start()
    for b in range(2):
        in_copy(b, b).wait()
        transpose_chunk(b)
        in_copy(b + 2, b).start()
        out_copy(b, b).start()

    def ring(i, carry):
        for b in range(2):
            g = 2 * i + b
            in_copy(g, b).wait()
            out_copy(g - 2, b).wait()
            transpose_chunk(b)
            in_copy(g + 2, b).start()
            out_copy(g, b).start()
        return carry

    lax.fori_loop(1, _TC_NCHUNK // 2 - 1, ring, 0)

    for b in range(2):
        g = _TC_NCHUNK - 2 + b
        in_copy(g, b).wait()
        out_copy(g - 2, b).wait()
        transpose_chunk(b)
        out_copy(g, b).start()
    for b in range(2):
        out_copy(_TC_NCHUNK - 2 + b, b).wait()


_PER_W = _N // _NW
_IDXROW = 128
_CPB = 5
_CHUNK = _CPB * _IDXROW
_NCHUNK = _PER_W // _CHUNK


@functools.partial(
    pl.kernel,
    mesh=_mesh,
    compiler_params=pltpu.CompilerParams(use_tc_tiling_on_sc=False, needs_layout_passes=False),
    out_type=jax.ShapeDtypeStruct((_D, _N), jnp.float32),
    scratch_types=[
        pltpu.VMEM((2, _CPB, _IDXROW), jnp.int32),
        pltpu.VMEM((2, _CHUNK, _D), jnp.float32),
        pltpu.VMEM((2, _D, _CHUNK), jnp.float32),
        pltpu.SemaphoreType.DMA((2,)),
        pltpu.SemaphoreType.DMA((2,)),
        pltpu.SemaphoreType.DMA((2,)),
    ],
)
def _gather_rows(idx_hbm, rm_hbm, out_hbm, idx_v, rows_v, rowsT_v, isem, gsem, osem):
    wid = lax.axis_index("s") * _NC + lax.axis_index("c")
    base = wid * _PER_W
    iota = lax.iota(jnp.int32, _LANES)

    def idx_copy(g, slot):
        return pltpu.make_async_copy(
            idx_hbm.at[wid, g], idx_v.at[slot], isem.at[slot])

    def out_copy(g, slot):
        return pltpu.make_async_copy(
            rowsT_v.at[slot],
            out_hbm.at[:, pl.ds(base + g * _CHUNK, _CHUNK)],
            osem.at[slot])

    def run_gathers(slot):
        handles = [
            pltpu.make_async_copy(
                rm_hbm.at[idx_v.at[slot, j]],
                rows_v.at[slot, pl.ds(j * _IDXROW, _IDXROW)],
                gsem.at[slot],
            )
            for j in range(_CPB)
        ]
        for h in handles:
            h.start()
        return handles

    def drain(handles):
        for h in handles:
            h.wait()

    def transpose_chunk(slot):
        def row_body(i, _):
            v0 = plsc.load_gather(rows_v.at[slot], [jnp.full((_LANES,), i, jnp.int32), iota])
            v1 = plsc.load_gather(rows_v.at[slot], [jnp.full((_LANES,), i, jnp.int32), iota + _LANES])
            plsc.store_scatter(rowsT_v.at[slot], [iota, jnp.full((_LANES,), i, jnp.int32)], v0)
            plsc.store_scatter(rowsT_v.at[slot], [iota + _LANES, jnp.full((_LANES,), i, jnp.int32)], v1)
            return 0
        lax.fori_loop(0, _CHUNK, row_body, 0, unroll=4)

    idx_copy(0, 0).start()
    idx_copy(1, 1).start()
    idx_copy(0, 0).wait()
    h = run_gathers(0)

    def step(g, slot, handles, last):
        if not last:
            idx_copy(g + 1, 1 - slot).wait()
        nxt = run_gathers(1 - slot) if not last else None
        drain(handles)
        if g + 2 < _NCHUNK:
            idx_copy(g + 2, slot).start()
        if g >= 2:
            out_copy(g - 2, slot).wait()
        transpose_chunk(slot)
        out_copy(g, slot).start()
        return nxt

    h = step(0, 0, h, last=False)
    h = step(1, 1, h, last=False)

    def ring(i, carry):
        for b in range(2):
            g = 2 * i + b
            idx_copy(g + 1, 1 - b).wait()
            nxt = run_gathers(1 - b)
            drain_handles = [
                pltpu.make_async_copy(
                    rm_hbm.at[idx_v.at[b, j]],
                    rows_v.at[b, pl.ds(j * _IDXROW, _IDXROW)],
                    gsem.at[b],
                )
                for j in range(_CPB)
            ]
            for hh in drain_handles:
                hh.wait()
            idx_copy(g + 2, b).start()
            out_copy(g - 2, b).wait()
            transpose_chunk(b)
            out_copy(g, b).start()
        return carry

    lax.fori_loop(1, _NCHUNK // 2 - 1, ring, 0)

    g0 = _NCHUNK - 2
    idx_copy(g0 + 1, 1).wait()
    h2 = run_gathers(1)
    drain(h)
    out_copy(g0 - 2, 0).wait()
    transpose_chunk(0)
    out_copy(g0, 0).start()
    drain(h2)
    out_copy(g0 - 1, 1).wait()
    transpose_chunk(1)
    out_copy(g0 + 1, 1).start()
    out_copy(g0, 0).wait()
    out_copy(g0 + 1, 1).wait()


def kernel(indices, table):
    idx4 = indices.reshape(_NW, _NCHUNK, _CPB, _IDXROW)
    rm = _transpose_table(table.T.reshape(_D, _TC_TOT, _TC_CHUNK))
    rowsT = _gather_rows(idx4, rm)
    all_embs = rowsT.T[:, None, :]
    boundaries = jnp.arange(_B + 1, dtype=jnp.int32) * jnp.int32(_L)
    return (all_embs, boundaries)


# speedup vs baseline: 4.0590x; 4.0590x over previous
import functools

import jax
import jax.numpy as jnp
from jax import lax
from jax.experimental import pallas as pl
from jax.experimental.pallas import tpu as pltpu
from jax.experimental.pallas import tpu_sc as plsc

_B, _L = 4096, 200
_N = _B * _L
_D = 32
_V = 1000000
_NC, _NS = 2, 16
_NW = _NC * _NS
_PER_W = _N // _NW
_IDXROW = 128
_CPB = 10
_CHUNK = _CPB * _IDXROW
_NCHUNK = _PER_W // _CHUNK
_NBUF = 2

_mesh = plsc.VectorSubcoreMesh(core_axis_name="c", subcore_axis_name="s")


@functools.partial(
    pl.kernel,
    mesh=_mesh,
    compiler_params=pltpu.CompilerParams(use_tc_tiling_on_sc=False),
    out_type=jax.ShapeDtypeStruct((_NW, _NCHUNK, _CHUNK, _D), jnp.float32),
    scratch_types=[
        pltpu.VMEM((_NBUF, _CPB, _IDXROW), jnp.int32),
        pltpu.VMEM((_NBUF, _CHUNK, _D), jnp.float32),
        pltpu.SemaphoreType.DMA((_NBUF,)),
        pltpu.SemaphoreType.DMA((_NBUF,)),
        pltpu.SemaphoreType.DMA((_NBUF,)),
    ],
)
def _gather_rows(idx_hbm, table_hbm, out_hbm, idx_v, rows_v, isem, gsem, osem):
    wid = lax.axis_index("s") * _NC + lax.axis_index("c")

    def idx_copy(g, slot):
        return pltpu.make_async_copy(
            idx_hbm.at[wid, g], idx_v.at[slot], isem.at[slot])

    def out_copy(g, slot):
        return pltpu.make_async_copy(
            rows_v.at[slot], out_hbm.at[wid, g], osem.at[slot])

    def run_gathers(slot):
        handles = [
            pltpu.make_async_copy(
                table_hbm.at[idx_v.at[slot, j]],
                rows_v.at[slot, pl.ds(j * _IDXROW, _IDXROW)],
                gsem.at[slot],
            )
            for j in range(_CPB)
        ]
        for h in handles:
            h.start()
        for h in handles:
            h.wait()

    idx_copy(0, 0).start()
    idx_copy(1, 1).start()

    for b in range(_NBUF):
        idx_copy(b, b).wait()
        run_gathers(b)
        idx_copy(b + _NBUF, b).start()
        out_copy(b, b).start()

    def ring(i, carry):
        for b in range(_NBUF):
            g = _NBUF * i + b
            idx_copy(g, b).wait()
            out_copy(g - _NBUF, b).wait()
            run_gathers(b)
            idx_copy(g + _NBUF, b).start()
            out_copy(g, b).start()
        return carry

    lax.fori_loop(1, _NCHUNK // _NBUF - 1, ring, 0)

    for b in range(_NBUF):
        g = _NCHUNK - _NBUF + b
        idx_copy(g, b).wait()
        out_copy(g - _NBUF, b).wait()
        run_gathers(b)
        out_copy(g, b).start()
    for b in range(_NBUF):
        out_copy(_NCHUNK - _NBUF + b, b).wait()


def kernel(indices, table):
    idx4 = (indices * 4).reshape(_NW, _NCHUNK, _CPB, _IDXROW)
    t128 = jnp.pad(table, ((0, 0), (0, 96)))
    rm = t128.reshape(_V * 4, _D)
    rows = _gather_rows(idx4, rm)
    all_embs = rows.reshape(_N, 1, _D)
    boundaries = jnp.arange(_B + 1, dtype=jnp.int32) * jnp.int32(_L)
    return (all_embs, boundaries)


# speedup vs baseline: 5.4844x; 1.3512x over previous
import functools

import jax
import jax.numpy as jnp
from jax import lax
from jax.experimental import pallas as pl
from jax.experimental.pallas import tpu as pltpu
from jax.experimental.pallas import tpu_sc as plsc

_B, _L = 4096, 200
_N = _B * _L
_D = 32
_V = 1000000
_NC, _NS = 2, 16
_NW = _NC * _NS
_PER_W = _N // _NW
_IDXROW = 128
_CPB = 10
_CHUNK = _CPB * _IDXROW
_NCHUNK = _PER_W // _CHUNK
_NBUF = 2

_mesh = plsc.VectorSubcoreMesh(core_axis_name="c", subcore_axis_name="s")


@functools.partial(
    pl.kernel,
    mesh=_mesh,
    compiler_params=pltpu.CompilerParams(use_tc_tiling_on_sc=False),
    out_type=jax.ShapeDtypeStruct((_NW, _NCHUNK, _CHUNK, 128), jnp.float32),
    scratch_types=[
        pltpu.VMEM((_NBUF, _CPB, _IDXROW), jnp.int32),
        pltpu.VMEM((_NBUF, _CHUNK, _D), jnp.float32),
        pltpu.SemaphoreType.DMA((_NBUF,)),
        pltpu.SemaphoreType.DMA((_NBUF,)),
        pltpu.SemaphoreType.DMA((_NBUF,)),
    ],
)
def _gather_rows(idx_hbm, table_hbm, out_hbm, idx_v, rows_v, isem, gsem, osem):
    wid = lax.axis_index("s") * _NC + lax.axis_index("c")

    def idx_copy(g, slot):
        return pltpu.make_async_copy(
            idx_hbm.at[wid, g], idx_v.at[slot], isem.at[slot])

    def out_copy(g, slot):
        return pltpu.make_async_copy(
            rows_v.at[slot], out_hbm.at[wid, g, :, pl.ds(0, _D)], osem.at[slot])

    def run_gathers(slot):
        handles = [
            pltpu.make_async_copy(
                table_hbm.at[idx_v.at[slot, j]],
                rows_v.at[slot, pl.ds(j * _IDXROW, _IDXROW)],
                gsem.at[slot],
            )
            for j in range(_CPB)
        ]
        for h in handles:
            h.start()
        for h in handles:
            h.wait()

    idx_copy(0, 0).start()
    idx_copy(1, 1).start()

    for b in range(_NBUF):
        idx_copy(b, b).wait()
        run_gathers(b)
        idx_copy(b + _NBUF, b).start()
        out_copy(b, b).start()

    def ring(i, carry):
        for b in range(_NBUF):
            g = _NBUF * i + b
            idx_copy(g, b).wait()
            out_copy(g - _NBUF, b).wait()
            run_gathers(b)
            idx_copy(g + _NBUF, b).start()
            out_copy(g, b).start()
        return carry

    lax.fori_loop(1, _NCHUNK // _NBUF - 1, ring, 0)

    for b in range(_NBUF):
        g = _NCHUNK - _NBUF + b
        idx_copy(g, b).wait()
        out_copy(g - _NBUF, b).wait()
        run_gathers(b)
        out_copy(g, b).start()
    for b in range(_NBUF):
        out_copy(_NCHUNK - _NBUF + b, b).wait()


def kernel(indices, table):
    idx4 = indices.reshape(_NW, _NCHUNK, _CPB, _IDXROW)
    rows128 = _gather_rows(idx4, table)
    all_embs = rows128.reshape(_N, 128)[:, : _D][:, None, :]
    boundaries = jnp.arange(_B + 1, dtype=jnp.int32) * jnp.int32(_L)
    return (all_embs, boundaries)


# speedup vs baseline: 6.0637x; 1.1056x over previous
import functools

import jax
import jax.numpy as jnp
from jax import lax
from jax.experimental import pallas as pl
from jax.experimental.pallas import tpu as pltpu
from jax.experimental.pallas import tpu_sc as plsc

_B, _L = 4096, 200
_N = _B * _L
_D = 32
_NC, _NS = 2, 16
_NW = _NC * _NS
_PER_W = _N // _NW
_IDXROW = 128
_CPB = 5
_CHUNK = _CPB * _IDXROW
_NCHUNK = _PER_W // _CHUNK
_NCI = _N // 128
_XT_L = 129

_mesh = plsc.VectorSubcoreMesh(core_axis_name="c", subcore_axis_name="s")


@functools.partial(
    pl.kernel,
    mesh=_mesh,
    compiler_params=pltpu.CompilerParams(
        use_tc_tiling_on_sc=False, needs_layout_passes=False),
    out_type=jax.ShapeDtypeStruct((4, _NCI, 8, 128), jnp.float32),
    scratch_types=[
        pltpu.VMEM((2, _CPB, _IDXROW), jnp.int32),
        pltpu.VMEM((2, _CHUNK, _D), jnp.float32),
        pltpu.VMEM((2, 4, _CPB, 8, _XT_L), jnp.float32),
        pltpu.SemaphoreType.DMA((2,)),
        pltpu.SemaphoreType.DMA((2,)),
        pltpu.SemaphoreType.DMA((2,)),
    ],
)
def _gather_rows(idx_hbm, table_hbm, out_hbm, idx_v, rows_v, xt_v, isem, gsem, osem):
    wid = lax.axis_index("s") * _NC + lax.axis_index("c")
    iota = lax.iota(jnp.int32, 16)
    rg_lo = iota >> 3
    s_vec = iota & 7

    def idx_copy(g, slot):
        return pltpu.make_async_copy(
            idx_hbm.at[wid, g], idx_v.at[slot], isem.at[slot])

    def out_copy(g, slot):
        return pltpu.make_async_copy(
            xt_v.at[slot, :, :, :, pl.ds(0, 128)],
            out_hbm.at[:, pl.ds(wid * _NCHUNK * _CPB + g * _CPB, _CPB), :, :],
            osem.at[slot])

    def gather_handles(slot):
        return [
            pltpu.make_async_copy(
                table_hbm.at[idx_v.at[slot, j]],
                rows_v.at[slot, pl.ds(j * _IDXROW, _IDXROW)],
                gsem.at[slot],
            )
            for j in range(_CPB)
        ]

    def start_gathers(slot):
        for h in gather_handles(slot):
            h.start()

    def drain_gathers(slot):
        for h in gather_handles(slot):
            h.wait()

    def transpose(slot):
        for ci in range(_CPB):
            civ = jnp.full((16,), ci, jnp.int32)

            def lbody(l, _, _ci=ci, _civ=civ):
                i = _ci * 128 + l
                lv = jnp.full((16,), l, jnp.int32)
                v0 = rows_v[slot, i, pl.ds(0, 16)]
                v1 = rows_v[slot, i, pl.ds(16, 16)]
                plsc.store_scatter(xt_v.at[slot], [rg_lo, _civ, s_vec, lv], v0)
                plsc.store_scatter(xt_v.at[slot], [rg_lo + 2, _civ, s_vec, lv], v1)
                return 0

            lax.fori_loop(0, 128, lbody, 0, unroll=4)

    def body(g, s, first_pair=False, no_osem=False, no_idx=False):
        o = 1 - s
        idx_copy(g, s).wait()
        start_gathers(s)
        if not first_pair:
            drain_gathers(o)
            if not no_idx:
                idx_copy(g + 1, o).start()
            if not no_osem:
                out_copy(g - 3, o).wait()
            transpose(o)
            out_copy(g - 1, o).start()

    idx_copy(0, 0).start()
    idx_copy(1, 1).start()
    body(0, 0, first_pair=True)
    body(1, 1, no_osem=True)
    body(2, 0, no_osem=True)
    body(3, 1)

    def ring(i, carry):
        for b in range(2):
            body(2 * i + b, b)
        return carry

    lax.fori_loop(2, _NCHUNK // 2 - 1, ring, 0)

    body(_NCHUNK - 2, 0)
    body(_NCHUNK - 1, 1, no_idx=True)

    drain_gathers(1)
    out_copy(_NCHUNK - 3, 1).wait()
    transpose(1)
    out_copy(_NCHUNK - 1, 1).start()
    out_copy(_NCHUNK - 2, 0).wait()
    out_copy(_NCHUNK - 1, 1).wait()


def kernel(indices, table):
    idx4 = indices.reshape(_NW, _NCHUNK, _CPB, _IDXROW)
    tiles = _gather_rows(idx4, table)
    all_embs = tiles.transpose(1, 3, 0, 2).reshape(_N, _D)[:, None, :]
    boundaries = jnp.arange(_B + 1, dtype=jnp.int32) * jnp.int32(_L)
    return (all_embs, boundaries)


# speedup vs baseline: 6.1184x; 1.0090x over previous
import functools

import jax
import jax.numpy as jnp
from jax import lax
from jax.experimental import pallas as pl
from jax.experimental.pallas import tpu as pltpu
from jax.experimental.pallas import tpu_sc as plsc

_B, _L = 4096, 200
_N = _B * _L
_D = 32
_NC, _NS = 2, 16
_NW = _NC * _NS
_PER_W = _N // _NW
_IDXROW = 128
_CPB = 5
_CHUNK = _CPB * _IDXROW
_NCHUNK = _PER_W // _CHUNK
_NCI = _N // 128
_XT_L = 129

_mesh = plsc.VectorSubcoreMesh(core_axis_name="c", subcore_axis_name="s")


@functools.partial(
    pl.kernel,
    mesh=_mesh,
    compiler_params=pltpu.CompilerParams(
        use_tc_tiling_on_sc=False, needs_layout_passes=False),
    out_type=jax.ShapeDtypeStruct((4, _NCI, 8, 128), jnp.float32),
    scratch_types=[
        pltpu.VMEM((2, _CPB, _IDXROW), jnp.int32),
        pltpu.VMEM((2, _CHUNK, _D), jnp.float32),
        pltpu.VMEM((2, 4, _CPB, 8, _XT_L), jnp.float32),
        pltpu.SemaphoreType.DMA((2,)),
        pltpu.SemaphoreType.DMA((2,)),
        pltpu.SemaphoreType.DMA((2,)),
    ],
)
def _gather_rows(idx_hbm, table_hbm, out_hbm, idx_v, rows_v, xt_v, isem, gsem, osem):
    wid = lax.axis_index("s") * _NC + lax.axis_index("c")
    iota = lax.iota(jnp.int32, 16)
    rg_lo = iota >> 3
    s_vec = iota & 7

    def idx_copy(g, slot):
        return pltpu.make_async_copy(
            idx_hbm.at[wid, g], idx_v.at[slot], isem.at[slot])

    def out_copy(g, slot):
        return pltpu.make_async_copy(
            xt_v.at[slot, :, :, :, pl.ds(0, 128)],
            out_hbm.at[:, pl.ds(wid * _NCHUNK * _CPB + g * _CPB, _CPB), :, :],
            osem.at[slot])

    def gather_handles(slot):
        return [
            pltpu.make_async_copy(
                table_hbm.at[idx_v.at[slot, j]],
                rows_v.at[slot, pl.ds(j * _IDXROW, _IDXROW)],
                gsem.at[slot],
            )
            for j in range(_CPB)
        ]

    def start_gathers(slot):
        for h in gather_handles(slot):
            h.start()

    def drain_gathers(slot):
        for h in gather_handles(slot):
            h.wait()

    def transpose(slot):
        for ci in range(_CPB):
            civ = jnp.full((16,), ci, jnp.int32)

            def lbody(l, _, _ci=ci, _civ=civ):
                i = _ci * 128 + l
                lv = jnp.full((16,), l, jnp.int32)
                v0 = rows_v[slot, i, pl.ds(0, 16)]
                v1 = rows_v[slot, i, pl.ds(16, 16)]
                plsc.store_scatter(xt_v.at[slot], [rg_lo, _civ, s_vec, lv], v0)
                plsc.store_scatter(xt_v.at[slot], [rg_lo + 2, _civ, s_vec, lv], v1)
                return 0

            lax.fori_loop(0, 128, lbody, 0, unroll=8)

    def body(g, s, first_pair=False, no_osem=False, no_idx=False):
        o = 1 - s
        idx_copy(g, s).wait()
        start_gathers(s)
        if not first_pair:
            drain_gathers(o)
            if not no_idx:
                idx_copy(g + 1, o).start()
            if not no_osem:
                out_copy(g - 3, o).wait()
            transpose(o)
            out_copy(g - 1, o).start()

    idx_copy(0, 0).start()
    idx_copy(1, 1).start()
    body(0, 0, first_pair=True)
    body(1, 1, no_osem=True)
    body(2, 0, no_osem=True)
    body(3, 1)

    def ring(i, carry):
        for b in range(2):
            body(2 * i + b, b)
        return carry

    lax.fori_loop(2, _NCHUNK // 2 - 1, ring, 0)

    body(_NCHUNK - 2, 0)
    body(_NCHUNK - 1, 1, no_idx=True)

    drain_gathers(1)
    out_copy(_NCHUNK - 3, 1).wait()
    transpose(1)
    out_copy(_NCHUNK - 1, 1).start()
    out_copy(_NCHUNK - 2, 0).wait()
    out_copy(_NCHUNK - 1, 1).wait()


def kernel(indices, table):
    idx4 = indices.reshape(_NW, _NCHUNK, _CPB, _IDXROW)
    tiles = _gather_rows(idx4, table)
    all_embs = tiles.transpose(1, 3, 0, 2).reshape(_N, _D)[:, None, :]
    boundaries = jnp.arange(_B + 1, dtype=jnp.int32) * jnp.int32(_L)
    return (all_embs, boundaries)
